# baseline (speedup 1.0000x reference)
import functools

import jax
import jax.numpy as jnp
from jax import lax
from jax.experimental import pallas as pl
from jax.experimental.pallas import tpu as pltpu
from jax.experimental.pallas import tpu_sc as plsc

N = 10000
E = 320000
D = 128
DG = 16
NC = 2
NS = 16
NW = NC * NS
EW = E // NW
C = 40
IT = EW // C
NB = 2
ZA = C
ZD = 200


def _head_body(p_ref, g_ref, wmp_ref, bmp_ref, w1_ref, b1_ref, w2_ref,
               b2_ref, o_ref):
    agg = p_ref[0] + p_ref[1]
    inv = 1.0 / jnp.maximum(g_ref[0, :, :1] + g_ref[1, :, :1], 1.0)
    h = jnp.maximum(
        jnp.dot(agg * inv, wmp_ref[...], preferred_element_type=jnp.float32)
        + bmp_ref[...], 0.0)
    h = jnp.maximum(
        jnp.dot(h, w1_ref[...], preferred_element_type=jnp.float32)
        + b1_ref[...], 0.0)
    o_ref[...] = (
        jnp.dot(h, w2_ref[...], preferred_element_type=jnp.float32)
        + b2_ref[...])


def _sc_body(ft_hbm, ei_hbm, agg_hbm, deg_hbm,
             sall, dall, r0b, r1b, onesb, zdeg, acc, deg, s0, s1, sd):
    rows = [r0b, r1b]
    sems = [s0, s1]
    cid = lax.axis_index("c")
    sid = lax.axis_index("s")
    w = cid * NS + sid

    def frow(r, carry):
        for c8 in range(D // 16):
            r0b[r, pl.ds(c8 * 16, 16)] = jnp.zeros((16,), jnp.float32)
        onesb[r, pl.ds(0, DG)] = jnp.ones((DG,), jnp.float32)
        return carry
    lax.fori_loop(0, C, frow, 0)

    def fzd(r, carry):
        zdeg[r, pl.ds(0, DG)] = jnp.zeros((DG,), jnp.float32)
        return carry
    lax.fori_loop(0, ZD, fzd, 0)

    def zacc(t, carry):
        j = t * NS + sid

        @pl.when(j < N // ZA)
        def _():
            pltpu.sync_copy(r0b, acc.at[pl.ds(j * ZA, ZA)])
        return carry
    lax.fori_loop(0, -(-(N // ZA) // NS), zacc, 0)

    def zdg(t, carry):
        j = t * NS + sid

        @pl.when(j < N // ZD)
        def _():
            pltpu.sync_copy(zdeg, deg.at[pl.ds(j * ZD, ZD)])
        return carry
    lax.fori_loop(0, -(-(N // ZD) // NS), zdg, 0)
    plsc.subcore_barrier()

    pltpu.sync_copy(ei_hbm.at[0, pl.ds(w * EW, EW)], sall)
    pltpu.sync_copy(ei_hbm.at[1, pl.ds(w * EW, EW)], dall)

    def sx(i):
        return sall.at[pl.ds(pl.multiple_of(i * C, 8), C)]

    def dx(i):
        return dall.at[pl.ds(pl.multiple_of(i * C, 8), C)]

    for b in range(NB):
        pltpu.async_copy(ft_hbm.at[sx(b)], rows[b], sems[b])
        pltpu.async_copy(onesb, deg.at[dx(b)], sd, add=True)

    def visit(i, b):
        pltpu.make_async_copy(ft_hbm.at[sx(i)], rows[b], sems[b]).wait()
        pltpu.sync_copy(rows[b], acc.at[dx(i)], add=True)
        pltpu.async_copy(ft_hbm.at[sx(i + NB)], rows[b], sems[b])
        pltpu.async_copy(onesb, deg.at[dx(i + NB)], sd, add=True)

    def step(t, carry):
        for b in range(NB):
            visit(t * NB + b, b)
        return carry
    lax.fori_loop(0, IT // NB - 1, step, 0)

    for b in range(NB):
        i = IT - NB + b
        pltpu.make_async_copy(ft_hbm.at[sx(i)], rows[b], sems[b]).wait()
        pltpu.sync_copy(rows[b], acc.at[dx(i)], add=True)

    def drain(i, carry):
        pltpu.make_async_copy(onesb, deg.at[dx(i)], sd).wait()
        return carry
    lax.fori_loop(0, IT, drain, 0)
    plsc.subcore_barrier()

    def wacc(t, carry):
        j = t * NS + sid

        @pl.when(j < N // ZA)
        def _():
            pltpu.sync_copy(acc.at[pl.ds(j * ZA, ZA)], r0b)
            pltpu.sync_copy(r0b, agg_hbm.at[cid, pl.ds(j * ZA, ZA)])
        return carry
    lax.fori_loop(0, -(-(N // ZA) // NS), wacc, 0)

    def wdg(t, carry):
        j = t * NS + sid

        @pl.when(j < N // ZD)
        def _():
            pltpu.sync_copy(deg.at[pl.ds(j * ZD, ZD)], zdeg)
            pltpu.sync_copy(zdeg, deg_hbm.at[cid, pl.ds(j * ZD, ZD)])
        return carry
    lax.fori_loop(0, -(-(N // ZD) // NS), wdg, 0)


_sc_aggregate = functools.partial(
    pl.kernel,
    out_type=(jax.ShapeDtypeStruct((NC, N, D), jnp.float32),
              jax.ShapeDtypeStruct((NC, N, DG), jnp.float32)),
    mesh=plsc.VectorSubcoreMesh(
        core_axis_name="c", subcore_axis_name="s",
        num_cores=NC, num_subcores=NS),
    scratch_types=(
        [pltpu.VMEM((EW,), jnp.int32)] * 2
        + [pltpu.VMEM((C, D), jnp.float32)] * NB
        + [pltpu.VMEM((C, DG), jnp.float32),
           pltpu.VMEM((ZD, DG), jnp.float32),
           pltpu.VMEM_SHARED((N, D), jnp.float32),
           pltpu.VMEM_SHARED((N, DG), jnp.float32)]
        + [pltpu.SemaphoreType.DMA] * (NB + 1)
    ),
    compiler_params=pltpu.CompilerParams(use_tc_tiling_on_sc=False),
)(_sc_body)


def kernel(features, edge_index, W_mp, b_mp, W1, b1, W2, b2):
    parts, degp = _sc_aggregate(features, edge_index)

    out = pl.pallas_call(
        _head_body,
        out_shape=jax.ShapeDtypeStruct((N, D), jnp.float32),
    )(parts, degp, W_mp, b_mp.reshape(1, D), W1, b1.reshape(1, D),
      W2, b2.reshape(1, D))
    return out


# speedup vs baseline: 1.3881x; 1.3881x over previous
import functools

import jax
import jax.numpy as jnp
from jax import lax
from jax.experimental import pallas as pl
from jax.experimental.pallas import tpu as pltpu
from jax.experimental.pallas import tpu_sc as plsc

N = 10000
E = 320000
D = 128
DG = 16
NC = 2
NS = 16
NW = NC * NS
EW = E // NW
C = 40
IT = EW // C
P = 2
PC = IT // P
NB = 5
ZA = C
ZD = 100


def _head_body(p_ref, g_ref, wmp_ref, bmp_ref, w1_ref, b1_ref, w2_ref,
               b2_ref, o_ref):
    agg = p_ref[0] + p_ref[1]
    inv = 1.0 / jnp.maximum(g_ref[0, :, :1] + g_ref[1, :, :1], 1.0)
    h = jnp.maximum(
        jnp.dot(agg * inv, wmp_ref[...], preferred_element_type=jnp.float32)
        + bmp_ref[...], 0.0)
    h = jnp.maximum(
        jnp.dot(h, w1_ref[...], preferred_element_type=jnp.float32)
        + b1_ref[...], 0.0)
    o_ref[...] = (
        jnp.dot(h, w2_ref[...], preferred_element_type=jnp.float32)
        + b2_ref[...])


def _sc_body(ft_hbm, ei_hbm, agg_hbm, deg_hbm,
             sall, dall, r0b, r1b, r2b, r3b, r4b, onesb, zdeg, acc, deg,
             s0, s1, s2, s3, s4, sd):
    rows = [r0b, r1b, r2b, r3b, r4b]
    sems = [s0, s1, s2, s3, s4]
    cid = lax.axis_index("c")
    sid = lax.axis_index("s")
    w = cid * NS + sid

    def frow(r, carry):
        for c8 in range(D // 16):
            r0b[r, pl.ds(c8 * 16, 16)] = jnp.zeros((16,), jnp.float32)
        onesb[r, pl.ds(0, DG)] = jnp.ones((DG,), jnp.float32)
        return carry
    lax.fori_loop(0, C, frow, 0)

    def fzd(r, carry):
        zdeg[r, pl.ds(0, DG)] = jnp.zeros((DG,), jnp.float32)
        return carry
    lax.fori_loop(0, ZD, fzd, 0)

    def zacc(t, carry):
        j = t * NS + sid

        @pl.when(j < N // ZA)
        def _():
            pltpu.sync_copy(r0b, acc.at[pl.ds(j * ZA, ZA)])
        return carry
    lax.fori_loop(0, -(-(N // ZA) // NS), zacc, 0)

    def zdg(t, carry):
        j = t * NS + sid

        @pl.when(j < N // ZD)
        def _():
            pltpu.sync_copy(zdeg, deg.at[pl.ds(j * ZD, ZD)])
        return carry
    lax.fori_loop(0, -(-(N // ZD) // NS), zdg, 0)
    plsc.subcore_barrier()

    def sx(i):
        return sall.at[pl.ds(i * C, C)]

    def dx(i):
        return dall.at[pl.ds(i * C, C)]

    for p in range(P):
        pltpu.sync_copy(ei_hbm.at[0, pl.ds(w * EW + p * PC * C, PC * C)],
                        sall)
        pltpu.sync_copy(ei_hbm.at[1, pl.ds(w * EW + p * PC * C, PC * C)],
                        dall)
        for b in range(NB):
            pltpu.async_copy(ft_hbm.at[sx(b)], rows[b], sems[b])
            pltpu.async_copy(onesb, deg.at[dx(b)], sd, add=True)

        def visit(i, b):
            pltpu.make_async_copy(ft_hbm.at[sx(i)], rows[b], sems[b]).wait()
            pltpu.sync_copy(rows[b], acc.at[dx(i)], add=True)
            pltpu.async_copy(ft_hbm.at[sx(i + NB)], rows[b], sems[b])
            pltpu.async_copy(onesb, deg.at[dx(i + NB)], sd, add=True)

        def step(t, carry):
            for b in range(NB):
                visit(t * NB + b, b)
            return carry
        lax.fori_loop(0, PC // NB - 1, step, 0)

        for b in range(NB):
            i = PC - NB + b
            pltpu.make_async_copy(ft_hbm.at[sx(i)], rows[b], sems[b]).wait()
            pltpu.sync_copy(rows[b], acc.at[dx(i)], add=True)

        def drain(i, carry):
            pltpu.make_async_copy(onesb, deg.at[dx(i)], sd).wait()
            return carry
        lax.fori_loop(0, PC, drain, 0)
    plsc.subcore_barrier()

    def wacc(t, carry):
        j = t * NS + sid

        @pl.when(j < N // ZA)
        def _():
            pltpu.sync_copy(acc.at[pl.ds(j * ZA, ZA)], r0b)
            pltpu.sync_copy(r0b, agg_hbm.at[cid, pl.ds(j * ZA, ZA)])
        return carry
    lax.fori_loop(0, -(-(N // ZA) // NS), wacc, 0)

    def wdg(t, carry):
        j = t * NS + sid

        @pl.when(j < N // ZD)
        def _():
            pltpu.sync_copy(deg.at[pl.ds(j * ZD, ZD)], zdeg)
            pltpu.sync_copy(zdeg, deg_hbm.at[cid, pl.ds(j * ZD, ZD)])
        return carry
    lax.fori_loop(0, -(-(N // ZD) // NS), wdg, 0)


_sc_aggregate = functools.partial(
    pl.kernel,
    out_type=(jax.ShapeDtypeStruct((NC, N, D), jnp.float32),
              jax.ShapeDtypeStruct((NC, N, DG), jnp.float32)),
    mesh=plsc.VectorSubcoreMesh(
        core_axis_name="c", subcore_axis_name="s",
        num_cores=NC, num_subcores=NS),
    scratch_types=(
        [pltpu.VMEM((PC * C,), jnp.int32)] * 2
        + [pltpu.VMEM((C, D), jnp.float32)] * NB
        + [pltpu.VMEM((C, DG), jnp.float32),
           pltpu.VMEM((ZD, DG), jnp.float32),
           pltpu.VMEM_SHARED((N, D), jnp.float32),
           pltpu.VMEM_SHARED((N, DG), jnp.float32)]
        + [pltpu.SemaphoreType.DMA] * (NB + 1)
    ),
    compiler_params=pltpu.CompilerParams(use_tc_tiling_on_sc=False),
)(_sc_body)


def kernel(features, edge_index, W_mp, b_mp, W1, b1, W2, b2):
    parts, degp = _sc_aggregate(features, edge_index)

    out = pl.pallas_call(
        _head_body,
        out_shape=jax.ShapeDtypeStruct((N, D), jnp.float32),
    )(parts, degp, W_mp, b_mp.reshape(1, D), W1, b1.reshape(1, D),
      W2, b2.reshape(1, D))
    return out
